# baseline (speedup 1.0000x reference)
import functools

import jax
import jax.numpy as jnp
from jax import lax
from jax.experimental import pallas as pl
from jax.experimental.pallas import tpu as pltpu
from jax.experimental.pallas import tpu_sc as plsc

_NW = 32
_LANES = 16


def _wid():
    return lax.axis_index("s") * 2 + lax.axis_index("c")


def _splat(value):
    return jnp.full((_LANES,), value, dtype=jnp.int32)


@functools.lru_cache(maxsize=None)
def _make_lookup(B, S, D, V, BC):
    assert BC == 128 and D == 64 and S % 8 == 0 and B % BC == 0 and V % 2 == 0
    chunks = B // BC
    n_tasks = (S // 8) * chunks
    assert n_tasks % _NW == 0
    n_g = BC // _LANES

    mesh = plsc.VectorSubcoreMesh(core_axis_name="c", subcore_axis_name="s")

    @functools.partial(
        pl.kernel,
        mesh=mesh,
        out_type=jax.ShapeDtypeStruct((S, D, B), jnp.float32),
        scratch_types=[
            pltpu.VMEM((8, BC), jnp.int32),
            pltpu.VMEM((8, BC), jnp.int32),
            pltpu.VMEM((8, BC), jnp.int32),
            pltpu.VMEM((BC, 128), jnp.float32),
            pltpu.VMEM((BC, 128), jnp.float32),
            pltpu.VMEM((BC, 128), jnp.float32),
            pltpu.VMEM((D, BC), jnp.float32),
            pltpu.VMEM((D, BC), jnp.float32),
            pltpu.VMEM((S * D,), jnp.float32),
            pltpu.SemaphoreType.DMA,
            pltpu.SemaphoreType.DMA,
            pltpu.SemaphoreType.DMA,
            pltpu.SemaphoreType.DMA,
            pltpu.SemaphoreType.DMA,
        ],
        compiler_params=pltpu.CompilerParams(needs_layout_passes=False),
    )
    def look(x_hbm, tok_hbm, pos_hbm, out_hbm, idx_v, idx2_v, cb_v, rows_a,
             rows_b, rows_c, out_a, out_b, pos_v, g0, g1, g2, o0, o1):
        wid = _wid()
        lane = jnp.arange(_LANES, dtype=jnp.int32)
        tl = [lane + (g * _LANES) for g in range(n_g)]
        zero = _splat(0)
        pltpu.sync_copy(pos_hbm, pos_v)
        rows = [rows_a, rows_b, rows_c]
        outs = [out_a, out_b]
        gsems = [g0, g1, g2]
        osems = [o0, o1]

        def task(j, carry):
            t = j * _NW + wid
            s_hi = t // chunks
            b0 = pl.multiple_of((t % chunks) * BC, 128)
            pltpu.sync_copy(x_hbm.at[s_hi, :, pl.ds(b0, BC)], idx_v)

            @plsc.parallel_loop(0, 8 * n_g, unroll=4)
            def halve(g):
                r = g // n_g
                gg = g % n_g
                q = gg * _LANES
                raw = idx_v[r, pl.ds(q, _LANES)]
                idx2_v[r, pl.ds(q, _LANES)] = raw >> 1
                cb_v[r, pl.ds(q, _LANES)] = ((raw & 1) << 6) | (
                    (lane + (gg * _LANES)) << 7
                )

            gathers = [None, None, None]
            for k in range(2):
                gathers[k] = pltpu.async_copy(
                    tok_hbm.at[idx2_v.at[k]], rows[k], gsems[k]
                )
            out_copies = [None, None]

            for s_lo in range(8):
                gbuf = s_lo % 3
                buf = s_lo % 2
                s = s_hi * 8 + s_lo
                gathers[gbuf].wait()
                if s_lo + 2 < 8:
                    nb = (s_lo + 2) % 3
                    gathers[nb] = pltpu.async_copy(
                        tok_hbm.at[idx2_v.at[s_lo + 2]],
                        rows[nb],
                        gsems[nb],
                    )
                if out_copies[buf] is not None:
                    out_copies[buf].wait()

                rbuf = rows[gbuf]
                obuf = outs[buf]
                sD = s * D

                @plsc.parallel_loop(0, D, unroll=8)
                def col(d):
                    pd = plsc.load_gather(pos_v, [_splat(sD + d)])
                    dv = _splat(d)
                    for g in range(n_g):
                        cbg = cb_v[s_lo, pl.ds(g * _LANES, _LANES)]
                        vec = plsc.load_gather(rbuf, [zero, cbg + dv])
                        obuf[d, pl.ds(g * _LANES, _LANES)] = vec + pd
                out_copies[buf] = pltpu.async_copy(
                    obuf, out_hbm.at[s, :, pl.ds(b0, BC)], osems[buf]
                )

            for cp in out_copies:
                cp.wait()
            return carry

        lax.fori_loop(0, n_tasks // _NW, task, 0)

    return look


def kernel(x, token_table, pos_table):
    B, S = x.shape
    V, D = token_table.shape
    tok2 = token_table.reshape(V // 2, 2 * D)
    pos_flat = pos_table.reshape(-1)
    out_t = _make_lookup(B, S, D, V, 128)(
        x.T.reshape(S // 8, 8, B).astype(jnp.int32), tok2, pos_flat
    )
    return out_t.transpose(2, 0, 1)


# speedup vs baseline: 1.0154x; 1.0154x over previous
import functools

import jax
import jax.numpy as jnp
from jax import lax
from jax.experimental import pallas as pl
from jax.experimental.pallas import tpu as pltpu
from jax.experimental.pallas import tpu_sc as plsc

_NW = 32
_LANES = 16


def _wid():
    return lax.axis_index("s") * 2 + lax.axis_index("c")


def _splat(value):
    return jnp.full((_LANES,), value, dtype=jnp.int32)


@functools.lru_cache(maxsize=None)
def _make_lookup(B, S, D, V, BC):
    SB = 1024 // BC
    assert BC % 128 == 0 and D == 64 and S % SB == 0 and B % BC == 0 and V % 2 == 0
    chunks = B // BC
    n_tasks = (S // SB) * chunks
    assert n_tasks % _NW == 0
    n_g = BC // _LANES

    mesh = plsc.VectorSubcoreMesh(core_axis_name="c", subcore_axis_name="s")

    @functools.partial(
        pl.kernel,
        mesh=mesh,
        out_type=jax.ShapeDtypeStruct((S, D, B), jnp.float32),
        scratch_types=[
            pltpu.VMEM((SB, BC), jnp.int32),
            pltpu.VMEM((SB, BC), jnp.int32),
            pltpu.VMEM((SB, BC), jnp.int32),
            pltpu.VMEM((BC, 128), jnp.float32),
            pltpu.VMEM((BC, 128), jnp.float32),
            pltpu.VMEM((D, BC), jnp.float32),
            pltpu.VMEM((D, BC), jnp.float32),
            pltpu.VMEM((S * D,), jnp.float32),
            pltpu.SemaphoreType.DMA,
            pltpu.SemaphoreType.DMA,
            pltpu.SemaphoreType.DMA,
            pltpu.SemaphoreType.DMA,
        ],
        compiler_params=pltpu.CompilerParams(needs_layout_passes=False),
    )
    def look(x_hbm, tok_hbm, pos_hbm, out_hbm, idx_v, idx2_v, cb_v, rows_a,
             rows_b, out_a, out_b, pos_v, g0, g1, o0, o1):
        wid = _wid()
        lane = jnp.arange(_LANES, dtype=jnp.int32)
        tl = [lane + (g * _LANES) for g in range(n_g)]
        zero = _splat(0)
        pltpu.sync_copy(pos_hbm, pos_v)
        rows = [rows_a, rows_b]
        outs = [out_a, out_b]
        gsems = [g0, g1]
        osems = [o0, o1]

        def task(j, carry):
            t = j * _NW + wid
            s_hi = t // chunks
            b0 = pl.multiple_of((t % chunks) * BC, 128)
            pltpu.sync_copy(x_hbm.at[s_hi, :, pl.ds(b0, BC)], idx_v)

            @plsc.parallel_loop(0, SB * n_g, unroll=4)
            def halve(g):
                r = g // n_g
                gg = g % n_g
                q = gg * _LANES
                raw = idx_v[r, pl.ds(q, _LANES)]
                idx2_v[r, pl.ds(q, _LANES)] = raw >> 1
                cb_v[r, pl.ds(q, _LANES)] = ((raw & 1) << 6) | (
                    (lane + (gg * _LANES)) << 7
                )

            gathers = [None, None]
            gathers[0] = [
                pltpu.async_copy(
                    tok_hbm.at[idx2_v.at[0, pl.ds(k * 128, 128)]],
                    rows[0].at[pl.ds(k * 128, 128)],
                    gsems[0],
                )
                for k in range(BC // 128)
            ]
            out_copies = [None, None]

            for s_lo in range(SB):
                buf = s_lo % 2
                s = s_hi * SB + s_lo
                for cp in gathers[buf]:
                    cp.wait()
                if s_lo + 1 < SB:
                    gathers[1 - buf] = [
                        pltpu.async_copy(
                            tok_hbm.at[idx2_v.at[s_lo + 1, pl.ds(k * 128, 128)]],
                            rows[1 - buf].at[pl.ds(k * 128, 128)],
                            gsems[1 - buf],
                        )
                        for k in range(BC // 128)
                    ]
                if out_copies[buf] is not None:
                    out_copies[buf].wait()

                rbuf = rows[buf]
                obuf = outs[buf]
                sD = s * D

                @plsc.parallel_loop(0, D, unroll=8)
                def col(d):
                    pd = plsc.load_gather(pos_v, [_splat(sD + d)])
                    dv = _splat(d)
                    for g in range(n_g):
                        cbg = cb_v[s_lo, pl.ds(g * _LANES, _LANES)]
                        vec = plsc.load_gather(rbuf, [zero, cbg + dv])
                        obuf[d, pl.ds(g * _LANES, _LANES)] = vec + pd
                out_copies[buf] = pltpu.async_copy(
                    obuf, out_hbm.at[s, :, pl.ds(b0, BC)], osems[buf]
                )

            for cp in out_copies:
                cp.wait()
            return carry

        lax.fori_loop(0, n_tasks // _NW, task, 0)

    return look


def kernel(x, token_table, pos_table):
    B, S = x.shape
    V, D = token_table.shape
    tok2 = token_table.reshape(V // 2, 2 * D)
    pos_flat = pos_table.reshape(-1)
    BC = 256
    out_t = _make_lookup(B, S, D, V, BC)(
        x.T.reshape(S // (1024 // BC), 1024 // BC, B).astype(jnp.int32),
        tok2,
        pos_flat,
    )
    return out_t.transpose(2, 0, 1)


# speedup vs baseline: 1.0522x; 1.0362x over previous
import functools

import jax
import jax.numpy as jnp
from jax import lax
from jax.experimental import pallas as pl
from jax.experimental.pallas import tpu as pltpu
from jax.experimental.pallas import tpu_sc as plsc

_NUM_WORKERS = 32


@functools.lru_cache(maxsize=None)
def _make_emb_kernel(B, S, D, NB):
    assert B % (_NUM_WORKERS * NB) == 0
    rows_per_w = B // _NUM_WORKERS
    n_chunks = rows_per_w // NB
    half = S // 2
    assert S % 2 == 0 and half <= 128 and D % 16 == 0

    mesh = plsc.VectorSubcoreMesh(core_axis_name="c", subcore_axis_name="s")

    @functools.partial(
        pl.kernel,
        mesh=mesh,
        out_type=jax.ShapeDtypeStruct((B * S, D), jnp.float32),
        scratch_types=[
            pltpu.VMEM((S, D), jnp.float32),
            pltpu.VMEM((NB * 2, half), jnp.int32),
            pltpu.VMEM((NB * S, D), jnp.float32),
            pltpu.SemaphoreType.DMA,
        ],
        compiler_params=pltpu.CompilerParams(use_tc_tiling_on_sc=False),
    )
    def emb(x_hbm, tok_hbm, pos_hbm, out_hbm, pos_v, idx_v, rows_v, sem):
        wid = lax.axis_index("s") * 2 + lax.axis_index("c")
        base = wid * rows_per_w
        pltpu.sync_copy(pos_hbm, pos_v)

        def chunk(ci, carry):
            b0 = base + ci * NB
            pltpu.sync_copy(x_hbm.at[pl.ds(b0 * 2, NB * 2)], idx_v)
            copies = [
                pltpu.async_copy(
                    tok_hbm.at[idx_v.at[t]],
                    rows_v.at[pl.ds(t * half, half)],
                    sem,
                )
                for t in range(NB * 2)
            ]
            for cp in copies:
                cp.wait()

            def add_row(j, c):
                for r in range(NB):
                    for q in range(D // 16):
                        sl = pl.ds(q * 16, 16)
                        rows_v[r * S + j, sl] = rows_v[r * S + j, sl] + pos_v[j, sl]
                return c

            lax.fori_loop(0, S, add_row, 0)
            pltpu.sync_copy(rows_v, out_hbm.at[pl.ds(b0 * S, NB * S)])
            return carry

        lax.fori_loop(0, n_chunks, chunk, 0)

    return emb


def kernel(x, token_table, pos_table):
    B, S = x.shape
    V, D = token_table.shape
    emb = _make_emb_kernel(B, S, D, 4)
    out = emb(x.reshape(B * 2, S // 2).astype(jnp.int32), token_table, pos_table)
    return out.reshape(B, S, D)
